# baseline (speedup 1.0000x reference)
import functools

import jax
import jax.numpy as jnp
from jax import lax
from jax.experimental import pallas as pl
from jax.experimental.pallas import tpu as pltpu
from jax.experimental.pallas import tpu_sc as plsc

NC, NS, L = 2, 16, 16
NW = NC * NS


def _pos_embed_sc(B, P, D):
    TOT = B * P
    PW = TOT // NW
    TPB = P // PW
    CH = 64
    NCH = PW // CH
    NV = PW // L
    mesh = plsc.VectorSubcoreMesh(core_axis_name="c", subcore_axis_name="s")

    @functools.partial(
        pl.kernel,
        out_type=jax.ShapeDtypeStruct((TOT, D), jnp.float32),
        mesh=mesh,
        scratch_types=[
            pltpu.VMEM((PW,), jnp.int32),
            pltpu.VMEM((PW,), jnp.float32),
            pltpu.VMEM((PW,), jnp.int32),
            pltpu.VMEM((CH, D), jnp.float32),
            pltpu.SemaphoreType.DMA,
        ],
        compiler_params=pltpu.CompilerParams(needs_layout_passes=False),
    )
    def k(mask_hbm, wpos_hbm, out_hbm, mbuf, maskf, posid, rows, sem):
        wid = lax.axis_index("s") * NC + lax.axis_index("c")
        base = wid * PW
        kk = wid % TPB
        rowbase = (wid // TPB) * P

        def pfx_outer(j, acc):
            pltpu.sync_copy(mask_hbm.at[pl.ds(rowbase + j * PW, PW)], mbuf)

            def pfx_inner(i, a):
                return a + mbuf[pl.ds(i * L, L)]

            return lax.fori_loop(0, NV, pfx_inner, acc)

        acc = lax.fori_loop(0, kk, pfx_outer, jnp.zeros((L,), jnp.int32))
        prefix = jnp.sum(acc)

        pltpu.sync_copy(mask_hbm.at[pl.ds(base, PW)], mbuf)

        def scan_body(i, carry):
            v = mbuf[pl.ds(i * L, L)]
            cs = plsc.cumsum(v) + carry
            posid[pl.ds(i * L, L)] = jnp.where(v > 0, cs - 1, 0)
            maskf[pl.ds(i * L, L)] = v.astype(jnp.float32)
            return carry + jnp.sum(v)

        lax.fori_loop(0, NV, scan_body, prefix)

        def chunk_body(cc, _):
            cbase = cc * CH
            pltpu.async_copy(
                wpos_hbm.at[posid.at[pl.ds(cbase, CH)]], rows, sem).wait()

            def row_body(r, _):
                mrep = plsc.load_gather(
                    maskf, [jnp.zeros((L,), jnp.int32) + (cbase + r)])
                for c in range(D // L):
                    rows[r, pl.ds(c * L, L)] = rows[r, pl.ds(c * L, L)] * mrep
                return 0

            lax.fori_loop(0, CH, row_body, 0)
            pltpu.sync_copy(rows, out_hbm.at[pl.ds(base + cbase, CH)])
            return 0

        lax.fori_loop(0, NCH, chunk_body, 0)

    return k


def kernel(tokens, past_kv_pos_offset, attention_mask, W_pos):
    B, P = attention_mask.shape
    _, D = W_pos.shape
    mask_flat = attention_mask.reshape(B * P).astype(jnp.int32)
    out = _pos_embed_sc(B, P, D)(mask_flat, W_pos)
    return out.reshape(B, P, D)
